# baseline (speedup 1.0000x reference)
import jax
import jax.numpy as jnp
from jax.experimental import pallas as pl
from jax.experimental.pallas import tpu as pltpu

N_ATOMS = 240
N_RES = 960
N = N_ATOMS + N_RES
T = 128
NT = 10
NP = NT * T
H = 64
N_LAYERS = 4
INV_SUBLAYERS = 2
INV_NORM = 1.0 / 100.0
N_BATCH = 8
F32 = jnp.float32


def _silu(v):
    return v * jax.nn.sigmoid(v)


def _row_ds(t):
    return pl.ds(pl.multiple_of(t * T, T), T)


def _fused_body(tmin, tmax, ha0, hr0, x0p, x0t3, bcol, brow3, marow,
                wa1, ba1, wa2, ba2, wr1, br1, wr2, br2, wemb, bemb,
                g_w1a, g_w1b, g_wcr, g_wcd, g_b1, g_w2, g_b2,
                g_wn1a, g_wn1b, g_bn1, g_wn2, g_bn2,
                e_c1a, e_c1b, e_wcr, e_wcd, e_b1, e_c2, e_b2, e_c3,
                wf, bf,
                out,
                h2d, a2d, b2d, x2d, xnew, xt3, rad, d0m, adjm):
    ea = _silu(ha0[...] @ wa1[...] + ba1[...]) @ wa2[...] + ba2[...]
    h2d[0:N_ATOMS, :] = ea @ wemb[...] + bemb[...]
    er = _silu(hr0[...] @ wr1[...] + br1[...]) @ wr2[...] + br2[...]
    h2d[N_ATOMS:N, :] = er @ wemb[...] + bemb[...]
    h2d[N:NP, :] = jnp.zeros((NP - N, H), F32)

    x2d[...] = x0p[...]
    xt3[...] = x0t3[...]

    def _overlap(a, b):
        return jnp.logical_and(tmax[a] >= tmin[b], tmax[b] >= tmin[a])

    def _pair_sq(xr, xt3r, a, b):
        acc = jnp.zeros((T, T), F32)
        for d in range(3):
            dx = xr[_row_ds(a), d:d + 1] - xt3r[b, d:d + 1, :]
            acc = acc + dx * dx
        return acc

    def _init_ab(k, _):
        a = k // NT
        b = k % NT

        @pl.when(_overlap(a, b))
        def _():
            bc = bcol[_row_ds(a), :]
            br = brow3[b]
            adjm[k] = jnp.where(bc == br, 1.0, 0.0).astype(F32)
            d0m[k] = _pair_sq(x0p, x0t3, a, b)
        return 0

    jax.lax.fori_loop(0, NT * NT, _init_ab, 0)

    def _refresh_rad():
        def _one(k, _):
            a = k // NT
            b = k % NT

            @pl.when(_overlap(a, b))
            def _():
                rad[k] = _pair_sq(x2d, xt3, a, b)
            return 0
        jax.lax.fori_loop(0, NT * NT, _one, 0)

    _refresh_rad()

    def _node_halves(w_a, w_b, bias):
        def _one(t, _):
            ht = h2d[_row_ds(t), :]
            a2d[_row_ds(t), :] = ht @ w_a + bias
            b2d[_row_ds(t), :] = ht @ w_b
            return 0
        jax.lax.fori_loop(0, NT, _one, 0)

    for blk in range(N_LAYERS):
        for sub in range(INV_SUBLAYERS):
            g = blk * INV_SUBLAYERS + sub
            w2 = g_w2[g]
            b2 = g_b2[g]
            wcr = g_wcr[g][None]
            wcd = g_wcd[g][None]
            wn1a = g_wn1a[g]
            wn1b = g_wn1b[g]
            bn1 = g_bn1[g]
            wn2 = g_wn2[g]
            bn2 = g_bn2[g]
            _node_halves(g_w1a[g], g_w1b[g], g_b1[g])

            def _gcl_i(a, _, w2=w2, b2=b2, wcr=wcr, wcd=wcd,
                       wn1a=wn1a, wn1b=wn1b, bn1=bn1, wn2=wn2, bn2=bn2):
                def _gcl_j(b, acc):
                    return jax.lax.cond(_overlap(a, b), _gcl_j_hit,
                                        lambda _, acc: acc, b, acc)

                def _gcl_j_hit(b, acc):
                    r_t = rad[b * NT + a]
                    d0_t = d0m[b * NT + a]
                    adj_t = adjm[b * NT + a]
                    bt = b2d[_row_ds(b), :]
                    at = a2d[_row_ds(a), :]
                    pre = (bt[:, None, :] + at[None, :, :])
                    m = _silu(pre).reshape(T * T, H)
                    m = _silu(m @ w2 + b2).reshape(T, T, H)
                    return acc + m.sum(axis=0)

                agg = jax.lax.fori_loop(0, NT, _gcl_j,
                                        jnp.zeros((T, H), F32)) * INV_NORM
                ht = h2d[_row_ds(a), :]
                o = _silu(ht @ wn1a + agg @ wn1b + bn1)
                o = o @ wn2 + bn2
                h2d[_row_ds(a), :] = ht + o
                return 0

            jax.lax.fori_loop(0, NT, _gcl_i, 0)

        c2 = e_c2[blk]
        eb2 = e_b2[blk]
        ewcr = e_wcr[blk][None]
        ewcd = e_wcd[blk][None]
        c3 = e_c3[blk][None]
        _node_halves(e_c1a[blk], e_c1b[blk], e_b1[blk])

        def _eq_i(a, _, c2=c2, eb2=eb2, ewcr=ewcr, ewcd=ewcd, c3=c3):
            def _eq_j(b, acc):
                return jax.lax.cond(_overlap(a, b), _eq_j_hit,
                                    lambda _, acc: acc, b, acc)

            def _eq_j_hit(b, acc):
                r_t = rad[a * NT + b]
                d0_t = d0m[a * NT + b]
                adj_t = adjm[a * NT + b]
                at = a2d[_row_ds(a), :]
                bt = b2d[_row_ds(b), :]
                pre = (at[:, None, :] + bt[None, :, :]
                       + r_t[:, :, None] * ewcr + d0_t[:, :, None] * ewcd)
                t2 = _silu(pre).reshape(T * T, H)
                t2 = _silu(t2 @ c2 + eb2).reshape(T, T, H)
                phi = (t2 * c3).sum(axis=2)
                s = phi * adj_t * (INV_NORM / jnp.sqrt(r_t + 1e-8))
                xi = x2d[_row_ds(a), :]
                xj = x2d[_row_ds(b), :]
                rs = s.sum(axis=1, keepdims=True)
                return acc + xi * rs - s @ xj

            xacc = jax.lax.fori_loop(0, NT, _eq_j, jnp.zeros((T, 128), F32))
            xnew[_row_ds(a), :] = x2d[_row_ds(a), :] + xacc
            return 0

        jax.lax.fori_loop(0, NT, _eq_i, 0)

        def _commit(t, _):
            xt = xnew[_row_ds(t), :]
            x2d[_row_ds(t), :] = xt
            xt3[t] = xt.T
            return 0

        jax.lax.fori_loop(0, NT, _commit, 0)
        if blk + 1 < N_LAYERS:
            _refresh_rad()

    hl = h2d[0:2 * T, :]
    seg = jax.lax.broadcasted_iota(jnp.int32, (N_BATCH, 1), 0)
    onehot = jnp.where(seg == marow[...], 1.0, 0.0).astype(F32)
    counts = onehot.sum(axis=1, keepdims=True)
    pooled = (onehot @ hl) / jnp.where(counts == 0.0, 1.0, counts)
    out[...] = _silu(pooled) @ wf[...] + bf[...]


def _stack(ps, key, field):
    return jnp.stack([p[key][field] for p in ps])


def kernel(xh_atoms, xh_residues, mask_atoms, mask_residues, params):
    f32 = F32
    ha0 = xh_atoms[:, 3:].astype(f32)
    hr0 = jnp.pad(xh_residues[:, 3:], ((0, 0), (0, 11))).astype(f32)
    x0 = jnp.concatenate([xh_atoms[:, :3], xh_residues[:, :3]], axis=0)
    x0p = jnp.pad(x0, ((0, NP - N), (0, 128 - 3))).astype(f32)
    x0t3 = x0p.reshape(NT, T, 128).transpose(0, 2, 1)

    bi = jnp.concatenate([mask_atoms, mask_residues]).astype(jnp.int32)
    batch = bi.astype(f32)
    bcol = jnp.pad(batch, (0, NP - N), constant_values=-1.0).reshape(NP, 1)
    brow3 = jnp.pad(batch, (0, NP - N), constant_values=-2.0).reshape(NT, 1, T)
    tmin = jnp.pad(bi, (0, NP - N), constant_values=127).reshape(NT, T).min(axis=1)
    tmax = jnp.pad(bi, (0, NP - N), constant_values=-1).reshape(NT, T).max(axis=1)
    marow = jnp.pad(mask_atoms.astype(jnp.int32), (0, 2 * T - N_ATOMS),
                    constant_values=-1).reshape(1, 2 * T)

    ae, re_, emb, fin = (params["atom_enc"], params["res_enc"],
                         params["embedding"], params["final"])
    wa1, ba1 = ae[0]["w"], ae[0]["b"].reshape(1, -1)
    wa2, ba2 = ae[1]["w"], ae[1]["b"].reshape(1, -1)
    wr1 = jnp.pad(re_[0]["w"], ((0, 11), (0, 0)))
    br1 = re_[0]["b"].reshape(1, -1)
    wr2, br2 = re_[1]["w"], re_[1]["b"].reshape(1, -1)
    wemb, bemb = emb["w"], emb["b"].reshape(1, -1)
    wf, bf = fin["w"], fin["b"].reshape(1, -1)

    gcls = [g for blk in params["blocks"] for g in blk["gcls"]]
    g_w1 = jnp.stack([g["edge1"]["w"] for g in gcls])
    g_w1a = g_w1[:, :H]
    g_w1b = g_w1[:, H:2 * H]
    g_wcr = g_w1[:, 2 * H:2 * H + 1]
    g_wcd = g_w1[:, 2 * H + 1:2 * H + 2]
    g_b1 = jnp.stack([g["edge1"]["b"] for g in gcls])[:, None]
    g_w2 = _stack(gcls, "edge2", "w")
    g_b2 = jnp.stack([g["edge2"]["b"] for g in gcls])[:, None]
    g_wn1 = _stack(gcls, "node1", "w")
    g_wn1a = g_wn1[:, :H]
    g_wn1b = g_wn1[:, H:]
    g_bn1 = jnp.stack([g["node1"]["b"] for g in gcls])[:, None]
    g_wn2 = _stack(gcls, "node2", "w")
    g_bn2 = jnp.stack([g["node2"]["b"] for g in gcls])[:, None]

    eqs = [blk["eq"] for blk in params["blocks"]]
    e_c1 = jnp.stack([e["c1"]["w"] for e in eqs])
    e_c1a = e_c1[:, :H]
    e_c1b = e_c1[:, H:2 * H]
    e_wcr = e_c1[:, 2 * H:2 * H + 1]
    e_wcd = e_c1[:, 2 * H + 1:2 * H + 2]
    e_b1 = jnp.stack([e["c1"]["b"] for e in eqs])[:, None]
    e_c2 = _stack(eqs, "c2", "w")
    e_b2 = jnp.stack([e["c2"]["b"] for e in eqs])[:, None]
    e_c3 = jnp.stack([e["c3"]["w"][:, 0] for e in eqs])[:, None]

    scratch = [
        pltpu.VMEM((NP, H), f32),
        pltpu.VMEM((NP, H), f32),
        pltpu.VMEM((NP, H), f32),
        pltpu.VMEM((NP, 128), f32),
        pltpu.VMEM((NP, 128), f32),
        pltpu.VMEM((NT, 128, T), f32),
        pltpu.VMEM((NT * NT, T, T), f32),
        pltpu.VMEM((NT * NT, T, T), f32),
        pltpu.VMEM((NT * NT, T, T), f32),
    ]

    n_vmem_in = 39
    return pl.pallas_call(
        _fused_body,
        out_shape=jax.ShapeDtypeStruct((N_BATCH, H), f32),
        in_specs=([pl.BlockSpec(memory_space=pltpu.SMEM)] * 2
                  + [pl.BlockSpec(memory_space=pltpu.VMEM)] * n_vmem_in),
        scratch_shapes=scratch,
        name="style_encoder_fused",
    )(tmin, tmax,
      ha0, hr0, x0p, x0t3, bcol, brow3, marow,
      wa1, ba1, wa2, ba2, wr1, br1, wr2, br2, wemb, bemb,
      g_w1a, g_w1b, g_wcr, g_wcd, g_b1, g_w2, g_b2,
      g_wn1a, g_wn1b, g_bn1, g_wn2, g_bn2,
      e_c1a, e_c1b, e_wcr, e_wcd, e_b1, e_c2, e_b2, e_c3,
      wf, bf)


# speedup vs baseline: 1.1609x; 1.1609x over previous
import jax
import jax.numpy as jnp
from jax.experimental import pallas as pl
from jax.experimental.pallas import tpu as pltpu

N_ATOMS = 240
N_RES = 960
N = N_ATOMS + N_RES
T = 64
NT = 20
NP = NT * T
H = 64
N_LAYERS = 4
INV_SUBLAYERS = 2
INV_NORM = 1.0 / 100.0
N_BATCH = 8
F32 = jnp.float32


def _silu(v):
    return v * jax.nn.sigmoid(v)


def _row_ds(t):
    return pl.ds(pl.multiple_of(t * T, T), T)


def _fused_body(tmin, tmax, ha0, hr0, x0p, x0t3, bcol, brow3, marow,
                wa1, ba1, wa2, ba2, wr1, br1, wr2, br2, wemb, bemb,
                g_w1a, g_w1b, g_wcr, g_wcd, g_b1, g_w2, g_b2,
                g_wn1a, g_wn1b, g_bn1, g_wn2, g_bn2,
                e_c1a, e_c1b, e_wcr, e_wcd, e_b1, e_c2, e_b2, e_c3,
                wf, bf,
                out,
                h2d, a2d, b2d, x2d, xnew, xt3, rad, d0m, adjm):
    ea = _silu(ha0[...] @ wa1[...] + ba1[...]) @ wa2[...] + ba2[...]
    h2d[0:N_ATOMS, :] = ea @ wemb[...] + bemb[...]
    er = _silu(hr0[...] @ wr1[...] + br1[...]) @ wr2[...] + br2[...]
    h2d[N_ATOMS:N, :] = er @ wemb[...] + bemb[...]
    h2d[N:NP, :] = jnp.zeros((NP - N, H), F32)

    x2d[...] = x0p[...]
    xt3[...] = x0t3[...]

    def _overlap(a, b):
        return jnp.logical_and(tmax[a] >= tmin[b], tmax[b] >= tmin[a])

    def _pair_sq(xr, xt3r, a, b):
        acc = jnp.zeros((T, T), F32)
        for d in range(3):
            dx = xr[_row_ds(a), d:d + 1] - xt3r[b, d:d + 1, :]
            acc = acc + dx * dx
        return acc

    def _init_ab(k, _):
        a = k // NT
        b = k % NT

        @pl.when(_overlap(a, b))
        def _():
            bc = bcol[_row_ds(a), :]
            br = brow3[b]
            adjm[k] = jnp.where(bc == br, 1.0, 0.0).astype(F32)
            d0m[k] = _pair_sq(x0p, x0t3, a, b)
        return 0

    jax.lax.fori_loop(0, NT * NT, _init_ab, 0)

    def _refresh_rad():
        def _one(k, _):
            a = k // NT
            b = k % NT

            @pl.when(_overlap(a, b))
            def _():
                rad[k] = _pair_sq(x2d, xt3, a, b)
            return 0
        jax.lax.fori_loop(0, NT * NT, _one, 0)

    _refresh_rad()

    def _node_halves(w_a, w_b, bias):
        def _one(t, _):
            ht = h2d[_row_ds(t), :]
            a2d[_row_ds(t), :] = ht @ w_a + bias
            b2d[_row_ds(t), :] = ht @ w_b
            return 0
        jax.lax.fori_loop(0, NT, _one, 0)

    for blk in range(N_LAYERS):
        for sub in range(INV_SUBLAYERS):
            g = blk * INV_SUBLAYERS + sub
            w2 = g_w2[g]
            b2 = g_b2[g]
            wcr = g_wcr[g][None]
            wcd = g_wcd[g][None]
            wn1a = g_wn1a[g]
            wn1b = g_wn1b[g]
            bn1 = g_bn1[g]
            wn2 = g_wn2[g]
            bn2 = g_bn2[g]
            _node_halves(g_w1a[g], g_w1b[g], g_b1[g])

            def _gcl_i(a, _, w2=w2, b2=b2, wcr=wcr, wcd=wcd,
                       wn1a=wn1a, wn1b=wn1b, bn1=bn1, wn2=wn2, bn2=bn2):
                def _gcl_j(b, acc):
                    return jax.lax.cond(_overlap(a, b), _gcl_j_hit,
                                        lambda _, acc: acc, b, acc)

                def _gcl_j_hit(b, acc):
                    r_t = rad[b * NT + a]
                    d0_t = d0m[b * NT + a]
                    adj_t = adjm[b * NT + a]
                    bt = b2d[_row_ds(b), :]
                    at = a2d[_row_ds(a), :]
                    pre = (bt[:, None, :] + at[None, :, :]
                           + r_t[:, :, None] * wcr + d0_t[:, :, None] * wcd)
                    m = _silu(pre).reshape(T * T, H)
                    m = _silu(m @ w2 + b2).reshape(T, T, H)
                    m = m * adj_t[:, :, None]
                    return acc + m.sum(axis=0)

                agg = jax.lax.fori_loop(0, NT, _gcl_j,
                                        jnp.zeros((T, H), F32)) * INV_NORM
                ht = h2d[_row_ds(a), :]
                o = _silu(ht @ wn1a + agg @ wn1b + bn1)
                o = o @ wn2 + bn2
                h2d[_row_ds(a), :] = ht + o
                return 0

            jax.lax.fori_loop(0, NT, _gcl_i, 0)

        c2 = e_c2[blk]
        eb2 = e_b2[blk]
        ewcr = e_wcr[blk][None]
        ewcd = e_wcd[blk][None]
        c3 = e_c3[blk][None]
        _node_halves(e_c1a[blk], e_c1b[blk], e_b1[blk])

        def _eq_i(a, _, c2=c2, eb2=eb2, ewcr=ewcr, ewcd=ewcd, c3=c3):
            def _eq_j(b, acc):
                return jax.lax.cond(_overlap(a, b), _eq_j_hit,
                                    lambda _, acc: acc, b, acc)

            def _eq_j_hit(b, acc):
                r_t = rad[a * NT + b]
                d0_t = d0m[a * NT + b]
                adj_t = adjm[a * NT + b]
                at = a2d[_row_ds(a), :]
                bt = b2d[_row_ds(b), :]
                pre = (at[:, None, :] + bt[None, :, :]
                       + r_t[:, :, None] * ewcr + d0_t[:, :, None] * ewcd)
                t2 = _silu(pre).reshape(T * T, H)
                t2 = _silu(t2 @ c2 + eb2).reshape(T, T, H)
                phi = (t2 * c3).sum(axis=2)
                s = phi * adj_t * (INV_NORM / jnp.sqrt(r_t + 1e-8))
                xi = x2d[_row_ds(a), :]
                xj = x2d[_row_ds(b), :]
                rs = s.sum(axis=1, keepdims=True)
                return acc + xi * rs - s @ xj

            xacc = jax.lax.fori_loop(0, NT, _eq_j, jnp.zeros((T, 128), F32))
            xnew[_row_ds(a), :] = x2d[_row_ds(a), :] + xacc
            return 0

        jax.lax.fori_loop(0, NT, _eq_i, 0)

        def _commit(t, _):
            xt = xnew[_row_ds(t), :]
            x2d[_row_ds(t), :] = xt
            xt3[t] = xt.T
            return 0

        jax.lax.fori_loop(0, NT, _commit, 0)
        if blk + 1 < N_LAYERS:
            _refresh_rad()

    hl = h2d[0:256, :]
    seg = jax.lax.broadcasted_iota(jnp.int32, (N_BATCH, 1), 0)
    onehot = jnp.where(seg == marow[...], 1.0, 0.0).astype(F32)
    counts = onehot.sum(axis=1, keepdims=True)
    pooled = (onehot @ hl) / jnp.where(counts == 0.0, 1.0, counts)
    out[...] = _silu(pooled) @ wf[...] + bf[...]


def _stack(ps, key, field):
    return jnp.stack([p[key][field] for p in ps])


def kernel(xh_atoms, xh_residues, mask_atoms, mask_residues, params):
    f32 = F32
    ha0 = xh_atoms[:, 3:].astype(f32)
    hr0 = jnp.pad(xh_residues[:, 3:], ((0, 0), (0, 11))).astype(f32)
    x0 = jnp.concatenate([xh_atoms[:, :3], xh_residues[:, :3]], axis=0)
    x0p = jnp.pad(x0, ((0, NP - N), (0, 128 - 3))).astype(f32)
    x0t3 = x0p.reshape(NT, T, 128).transpose(0, 2, 1)

    bi = jnp.concatenate([mask_atoms, mask_residues]).astype(jnp.int32)
    batch = bi.astype(f32)
    bcol = jnp.pad(batch, (0, NP - N), constant_values=-1.0).reshape(NP, 1)
    brow3 = jnp.pad(batch, (0, NP - N), constant_values=-2.0).reshape(NT, 1, T)
    tmin = jnp.pad(bi, (0, NP - N), constant_values=127).reshape(NT, T).min(axis=1)
    tmax = jnp.pad(bi, (0, NP - N), constant_values=-1).reshape(NT, T).max(axis=1)
    marow = jnp.pad(mask_atoms.astype(jnp.int32), (0, 256 - N_ATOMS),
                    constant_values=-1).reshape(1, 256)

    ae, re_, emb, fin = (params["atom_enc"], params["res_enc"],
                         params["embedding"], params["final"])
    wa1, ba1 = ae[0]["w"], ae[0]["b"].reshape(1, -1)
    wa2, ba2 = ae[1]["w"], ae[1]["b"].reshape(1, -1)
    wr1 = jnp.pad(re_[0]["w"], ((0, 11), (0, 0)))
    br1 = re_[0]["b"].reshape(1, -1)
    wr2, br2 = re_[1]["w"], re_[1]["b"].reshape(1, -1)
    wemb, bemb = emb["w"], emb["b"].reshape(1, -1)
    wf, bf = fin["w"], fin["b"].reshape(1, -1)

    gcls = [g for blk in params["blocks"] for g in blk["gcls"]]
    g_w1 = jnp.stack([g["edge1"]["w"] for g in gcls])
    g_w1a = g_w1[:, :H]
    g_w1b = g_w1[:, H:2 * H]
    g_wcr = g_w1[:, 2 * H:2 * H + 1]
    g_wcd = g_w1[:, 2 * H + 1:2 * H + 2]
    g_b1 = jnp.stack([g["edge1"]["b"] for g in gcls])[:, None]
    g_w2 = _stack(gcls, "edge2", "w")
    g_b2 = jnp.stack([g["edge2"]["b"] for g in gcls])[:, None]
    g_wn1 = _stack(gcls, "node1", "w")
    g_wn1a = g_wn1[:, :H]
    g_wn1b = g_wn1[:, H:]
    g_bn1 = jnp.stack([g["node1"]["b"] for g in gcls])[:, None]
    g_wn2 = _stack(gcls, "node2", "w")
    g_bn2 = jnp.stack([g["node2"]["b"] for g in gcls])[:, None]

    eqs = [blk["eq"] for blk in params["blocks"]]
    e_c1 = jnp.stack([e["c1"]["w"] for e in eqs])
    e_c1a = e_c1[:, :H]
    e_c1b = e_c1[:, H:2 * H]
    e_wcr = e_c1[:, 2 * H:2 * H + 1]
    e_wcd = e_c1[:, 2 * H + 1:2 * H + 2]
    e_b1 = jnp.stack([e["c1"]["b"] for e in eqs])[:, None]
    e_c2 = _stack(eqs, "c2", "w")
    e_b2 = jnp.stack([e["c2"]["b"] for e in eqs])[:, None]
    e_c3 = jnp.stack([e["c3"]["w"][:, 0] for e in eqs])[:, None]

    scratch = [
        pltpu.VMEM((NP, H), f32),
        pltpu.VMEM((NP, H), f32),
        pltpu.VMEM((NP, H), f32),
        pltpu.VMEM((NP, 128), f32),
        pltpu.VMEM((NP, 128), f32),
        pltpu.VMEM((NT, 128, T), f32),
        pltpu.VMEM((NT * NT, T, T), f32),
        pltpu.VMEM((NT * NT, T, T), f32),
        pltpu.VMEM((NT * NT, T, T), f32),
    ]

    n_vmem_in = 39
    return pl.pallas_call(
        _fused_body,
        out_shape=jax.ShapeDtypeStruct((N_BATCH, H), f32),
        in_specs=([pl.BlockSpec(memory_space=pltpu.SMEM)] * 2
                  + [pl.BlockSpec(memory_space=pltpu.VMEM)] * n_vmem_in),
        scratch_shapes=scratch,
        name="style_encoder_fused",
    )(tmin, tmax,
      ha0, hr0, x0p, x0t3, bcol, brow3, marow,
      wa1, ba1, wa2, ba2, wr1, br1, wr2, br2, wemb, bemb,
      g_w1a, g_w1b, g_wcr, g_wcd, g_b1, g_w2, g_b2,
      g_wn1a, g_wn1b, g_bn1, g_wn2, g_bn2,
      e_c1a, e_c1b, e_wcr, e_wcd, e_b1, e_c2, e_b2, e_c3,
      wf, bf)
